# Initial kernel scaffold; baseline (speedup 1.0000x reference)
import jax
import jax.numpy as jnp
from jax.experimental import pallas as pl


def kernel(x, Wqkv, bqkv, rpb, Wproj, bproj):
    raise NotImplementedError("write your pallas kernel here")


# speedup vs baseline: 2.9537x; 2.9537x over previous
import functools

import jax
import jax.numpy as jnp
from jax.experimental import pallas as pl
from jax.experimental.pallas import tpu as pltpu

N = 65536
DIM = 256
H = 8
WS = 64
C = DIM // H
SCALE = C ** -0.5

BLK = 2048
WPB = BLK // WS


def _fused_kernel(x_ref, wqkv_ref, bqkv_ref, rpb_ref, wproj_ref, bproj_ref,
                  out_ref):
    x = x_ref[...]
    qkv = jax.lax.dot_general(
        x, wqkv_ref[...], (((1,), (0,)), ((), ())),
        preferred_element_type=jnp.float32)
    qkv = qkv + bqkv_ref[...]

    outs = []
    for h in range(H):
        q = qkv[:, h * C:(h + 1) * C]
        k = qkv[:, DIM + h * C:DIM + (h + 1) * C]
        v = qkv[:, 2 * DIM + h * C:2 * DIM + (h + 1) * C]
        q3 = q.reshape(WPB, WS, C) * SCALE
        k3 = k.reshape(WPB, WS, C)
        v3 = v.reshape(WPB, WS, C)
        attn = jax.lax.dot_general(
            q3, k3, (((2,), (2,)), ((0,), (0,))),
            preferred_element_type=jnp.float32)
        attn = attn + rpb_ref[h][None]
        attn = attn - jnp.max(attn, axis=-1, keepdims=True)
        e = jnp.exp(attn)
        p = e / jnp.sum(e, axis=-1, keepdims=True)
        o3 = jax.lax.dot_general(
            p, v3, (((2,), (1,)), ((0,), (0,))),
            preferred_element_type=jnp.float32)
        outs.append(o3.reshape(BLK, C))
    o = jnp.concatenate(outs, axis=1)
    out = jax.lax.dot_general(
        o, wproj_ref[...], (((1,), (0,)), ((), ())),
        preferred_element_type=jnp.float32)
    out_ref[...] = out + bproj_ref[...]


@functools.partial(jax.jit, static_argnames=())
def kernel(x, Wqkv, bqkv, rpb, Wproj, bproj):
    n, dim = x.shape
    grid = (n // BLK,)
    return pl.pallas_call(
        _fused_kernel,
        grid=grid,
        in_specs=[
            pl.BlockSpec((BLK, dim), lambda i: (i, 0)),
            pl.BlockSpec((dim, 3 * dim), lambda i: (0, 0)),
            pl.BlockSpec((1, 3 * dim), lambda i: (0, 0)),
            pl.BlockSpec((H, WS, WS), lambda i: (0, 0, 0)),
            pl.BlockSpec((dim, dim), lambda i: (0, 0)),
            pl.BlockSpec((1, dim), lambda i: (0, 0)),
        ],
        out_specs=pl.BlockSpec((BLK, dim), lambda i: (i, 0)),
        out_shape=jax.ShapeDtypeStruct((n, dim), x.dtype),
        compiler_params=pltpu.CompilerParams(
            dimension_semantics=("arbitrary",),
        ),
    )(x, Wqkv, bqkv.reshape(1, 3 * dim), rpb, Wproj, bproj.reshape(1, dim))
